# Initial kernel scaffold; baseline (speedup 1.0000x reference)
import functools

import jax
import jax.numpy as jnp
from jax import lax
from jax.experimental import pallas as pl
from jax.experimental.pallas import tpu as pltpu
from jax.experimental.pallas import tpu_sc as plsc

NUM_E = 1024
DIM = 64
BATCH = 16384
BETA = 0.25

TILE = 512
GRID = BATCH // TILE

SC_CORES = 2
SC_SUBCORES = 16
SC_WORKERS = SC_CORES * SC_SUBCORES
ROWS_PER_WORKER = BATCH // SC_WORKERS


def _vq_argmin_kernel(z_ref, w_ref, idx_ref, msum_ref):
    i = pl.program_id(0)
    z = z_ref[...]
    w = w_ref[...]
    znorm = jnp.sum(z * z, axis=1, keepdims=True)
    wnorm = jnp.sum(w * w, axis=1)
    t = 2.0 * lax.dot_general(z, w, (((1,), (1,)), ((), ())))
    d = (znorm + wnorm[None, :]) - t
    m = jnp.min(d, axis=1, keepdims=True)
    col = lax.broadcasted_iota(jnp.int32, d.shape, 1)
    idx = jnp.min(jnp.where(d == m, col, NUM_E), axis=1)
    idx_ref[0, 0, :] = idx

    @pl.when(i == 0)
    def _init():
        msum_ref[0, 0] = 0.0

    msum_ref[0, 0] += jnp.sum(m)


def _tc_argmin(z, w):
    return pl.pallas_call(
        _vq_argmin_kernel,
        grid=(GRID,),
        in_specs=[
            pl.BlockSpec((TILE, DIM), lambda i: (i, 0)),
            pl.BlockSpec((NUM_E, DIM), lambda i: (0, 0)),
        ],
        out_specs=[
            pl.BlockSpec((1, 1, TILE), lambda i: (i, 0, 0)),
            pl.BlockSpec(memory_space=pltpu.SMEM),
        ],
        out_shape=[
            jax.ShapeDtypeStruct((GRID, 1, TILE), jnp.int32),
            jax.ShapeDtypeStruct((1, 1), jnp.float32),
        ],
        compiler_params=pltpu.CompilerParams(
            dimension_semantics=("arbitrary",),
        ),
    )(z, w)


@functools.partial(
    pl.kernel,
    mesh=plsc.VectorSubcoreMesh(core_axis_name="c", subcore_axis_name="s"),
    out_type=jax.ShapeDtypeStruct((BATCH, DIM), jnp.float32),
    scratch_types=[
        pltpu.VMEM((ROWS_PER_WORKER,), jnp.int32),
        pltpu.VMEM((ROWS_PER_WORKER, DIM), jnp.float32),
        pltpu.SemaphoreType.DMA,
    ],
)
def _sc_gather(table_hbm, idx_hbm, out_hbm, idx_v, rows_v, sem):
    wid = lax.axis_index("s") * SC_CORES + lax.axis_index("c")
    base = wid * ROWS_PER_WORKER
    pltpu.sync_copy(idx_hbm.at[pl.ds(base, ROWS_PER_WORKER)], idx_v)
    pltpu.async_copy(table_hbm.at[idx_v], rows_v, sem).wait()
    pltpu.sync_copy(rows_v, out_hbm.at[pl.ds(base, ROWS_PER_WORKER)])


def kernel(z, embedding_weight):
    idx3, msum = _tc_argmin(z, embedding_weight)
    idx = idx3.reshape(BATCH)
    z_q = _sc_gather(embedding_weight, idx)
    vq_loss = jnp.reshape(msum * ((1.0 + BETA) / (BATCH * DIM)), ())
    return (z_q, vq_loss)


# speedup vs baseline: 1.1587x; 1.1587x over previous
import functools

import jax
import jax.numpy as jnp
from jax import lax
from jax.experimental import pallas as pl
from jax.experimental.pallas import tpu as pltpu
from jax.experimental.pallas import tpu_sc as plsc

NUM_E = 1024
DIM = 64
BATCH = 16384
BETA = 0.25

TILE = 512
GRID = BATCH // TILE

SC_CORES = 2
SC_SUBCORES = 16
SC_WORKERS = SC_CORES * SC_SUBCORES
ROWS_PER_WORKER = BATCH // SC_WORKERS


def _vq_argmin_kernel(z_ref, w_ref, idx_ref, msum_ref):
    i = pl.program_id(0)
    z = z_ref[...]
    w = w_ref[...]
    znorm = jnp.sum(z * z, axis=1, keepdims=True)
    wnorm = jnp.sum(w * w, axis=1)
    t = 2.0 * lax.dot_general(z, w, (((1,), (1,)), ((), ())))
    d = (znorm + wnorm[None, :]) - t
    m = jnp.min(d, axis=1, keepdims=True)
    col = lax.broadcasted_iota(jnp.int32, d.shape, 1)
    idx = jnp.min(jnp.where(d == m, col, NUM_E), axis=1)
    idx_ref[0, 0, :] = idx

    @pl.when(i == 0)
    def _init():
        msum_ref[0, 0] = 0.0

    msum_ref[0, 0] += jnp.sum(m)


def _tc_argmin(z, w):
    return pl.pallas_call(
        _vq_argmin_kernel,
        grid=(GRID,),
        in_specs=[
            pl.BlockSpec((TILE, DIM), lambda i: (i, 0)),
            pl.BlockSpec((NUM_E, DIM), lambda i: (0, 0)),
        ],
        out_specs=[
            pl.BlockSpec((1, 1, TILE), lambda i: (i, 0, 0)),
            pl.BlockSpec(memory_space=pltpu.SMEM),
        ],
        out_shape=[
            jax.ShapeDtypeStruct((GRID, 1, TILE), jnp.int32),
            jax.ShapeDtypeStruct((1, 1), jnp.float32),
        ],
        compiler_params=pltpu.CompilerParams(
            dimension_semantics=("arbitrary",),
        ),
    )(z, w)


@functools.cache
def _make_sc_gather():
    @functools.partial(
        pl.kernel,
        mesh=plsc.VectorSubcoreMesh(core_axis_name="c", subcore_axis_name="s"),
        out_type=jax.ShapeDtypeStruct((BATCH, DIM), jnp.float32),
        scratch_types=[
            pltpu.VMEM((ROWS_PER_WORKER,), jnp.int32),
            pltpu.VMEM((ROWS_PER_WORKER, DIM), jnp.float32),
            pltpu.SemaphoreType.DMA,
        ],
        compiler_params=pltpu.CompilerParams(use_tc_tiling_on_sc=False),
    )
    def _sc_gather(table_hbm, idx_hbm, out_hbm, idx_v, rows_v, sem):
        wid = lax.axis_index("s") * SC_CORES + lax.axis_index("c")
        base = wid * ROWS_PER_WORKER
        pltpu.sync_copy(idx_hbm.at[pl.ds(base, ROWS_PER_WORKER)], idx_v)
        pltpu.async_copy(table_hbm.at[idx_v], rows_v, sem).wait()
        pltpu.sync_copy(rows_v, out_hbm.at[pl.ds(base, ROWS_PER_WORKER)])

    return _sc_gather


def kernel(z, embedding_weight):
    idx3, msum = _tc_argmin(z, embedding_weight)
    idx = idx3.reshape(BATCH)
    z_q = _make_sc_gather()(embedding_weight, idx)
    vq_loss = jnp.reshape(msum * ((1.0 + BETA) / (BATCH * DIM)), ())
    return (z_q, vq_loss)
